# Initial kernel scaffold; baseline (speedup 1.0000x reference)
import jax
import jax.numpy as jnp
from jax.experimental import pallas as pl


def kernel(inputs, W_gih, b_gih, W_glhh, W_grhh, W_cih, b_cih, W_clhh, W_crhh):
    raise NotImplementedError("write your pallas kernel here")


# speedup vs baseline: 6.9447x; 6.9447x over previous
import jax
import jax.numpy as jnp
from jax.experimental import pallas as pl

D = 12
N = 2 ** D - 1
B = 32
IN_DIM = 128
H = 128


def _dot(a, b):
    return jax.lax.dot_general(
        a, b, dimension_numbers=(((1,), (1,)), ((), ())),
        preferred_element_type=jnp.float32)


def _tree_gru_kernel(x_ref, wxc_ref, bxc_ref, wglr_ref, wclr_ref,
                     out_ref, outt_ref):
    wxc = wxc_ref[...]
    bxc = bxc_ref[...]
    wglr = wglr_ref[...]
    wclr = wclr_ref[...]

    n = 2 ** (D - 1)
    start = n - 1
    x = x_ref[0, start:start + n, :]
    a = _dot(x, wxc[4 * H:, :]) + bxc[:, 4 * H:]
    hidden = jax.nn.sigmoid(a[:, :H]) * jnp.tanh(a[:, H:])
    out_ref[0, start - 1:start - 1 + n, :] = hidden

    for level in range(D - 2, -1, -1):
        n = 2 ** level
        start = n - 1
        x = x_ref[0, start:start + n, :]
        ch = out_ref[0, 2 * start:2 * start + 2 * n, :]
        ch2 = ch.reshape(n, 2 * H)
        a = _dot(x, wxc) + bxc
        gates = jax.nn.sigmoid(a[:, :5 * H] + _dot(ch2, wglr))
        gated2 = gates[:, :2 * H] * ch2
        cell = jnp.tanh(a[:, 5 * H:] + _dot(gated2, wclr))
        z2 = gates[:, 2 * H:4 * H] * ch2
        hidden = z2[:, :H] + z2[:, H:] + gates[:, 4 * H:] * cell
        if level > 0:
            out_ref[0, start - 1:start - 1 + n, :] = hidden
        else:
            out_ref[0, N - 1:N, :] = hidden
            outt_ref[0] = hidden


def kernel(inputs, W_gih, b_gih, W_glhh, W_grhh, W_cih, b_cih, W_clhh, W_crhh):
    xt = jnp.transpose(inputs, (1, 0, 2))
    wxc = jnp.concatenate([W_gih, W_cih], axis=0)
    bxc = jnp.concatenate([b_gih, b_cih]).reshape(1, 6 * H)
    wglr = jnp.concatenate([W_glhh, W_grhh], axis=1)
    wclr = jnp.concatenate([W_clhh, W_crhh], axis=1)

    outputs, output_t = pl.pallas_call(
        _tree_gru_kernel,
        grid=(B,),
        in_specs=[
            pl.BlockSpec((1, N, IN_DIM), lambda b: (b, 0, 0)),
            pl.BlockSpec((6 * H, IN_DIM), lambda b: (0, 0)),
            pl.BlockSpec((1, 6 * H), lambda b: (0, 0)),
            pl.BlockSpec((5 * H, 2 * H), lambda b: (0, 0)),
            pl.BlockSpec((H, 2 * H), lambda b: (0, 0)),
        ],
        out_specs=[
            pl.BlockSpec((1, N, H), lambda b: (b, 0, 0)),
            pl.BlockSpec((1, 1, H), lambda b: (b, 0, 0)),
        ],
        out_shape=[
            jax.ShapeDtypeStruct((B, N, H), jnp.float32),
            jax.ShapeDtypeStruct((B, 1, H), jnp.float32),
        ],
    )(xt, wxc, bxc, wglr, wclr)
    return outputs, output_t.reshape(B, H)


# speedup vs baseline: 8.8392x; 1.2728x over previous
import jax
import jax.numpy as jnp
from jax.experimental import pallas as pl
from jax.experimental.pallas import tpu as pltpu

D = 12
N = 2 ** D - 1
B = 32
IN_DIM = 128
H = 128


def _dot(a, b):
    return jax.lax.dot_general(
        a, b, dimension_numbers=(((1,), (1,)), ((), ())),
        preferred_element_type=jnp.float32)


def _tree_gru_kernel(x_hbm, wxc_ref, bxc_ref, wglr_ref, wclr_ref,
                     out_ref, outt_ref, xbuf, sems):
    b = pl.program_id(0)

    def x_copy(slot, bidx):
        return pltpu.make_async_copy(
            x_hbm.at[:, bidx, :], xbuf.at[slot], sems.at[slot])

    @pl.when(b == 0)
    def _():
        x_copy(0, 0).start()

    @pl.when(b + 1 < B)
    def _():
        x_copy((b + 1) % 2, b + 1).start()

    slot = b % 2
    x_copy(slot, b).wait()
    x_ref = xbuf.at[slot]

    wxc = wxc_ref[...]
    bxc = bxc_ref[...]
    wglr = wglr_ref[...]
    wclr = wclr_ref[...]

    n = 2 ** (D - 1)
    start = n - 1
    x = x_ref[start:start + n, :]
    a = _dot(x, wxc[4 * H:, :]) + bxc[:, 4 * H:]
    hidden = jax.nn.sigmoid(a[:, :H]) * jnp.tanh(a[:, H:])
    out_ref[0, start - 1:start - 1 + n, :] = hidden

    for level in range(D - 2, -1, -1):
        n = 2 ** level
        start = n - 1
        x = x_ref[start:start + n, :]
        ch = out_ref[0, 2 * start:2 * start + 2 * n, :]
        ch2 = ch.reshape(n, 2 * H)
        a = _dot(x, wxc) + bxc
        gates = jax.nn.sigmoid(a[:, :5 * H] + _dot(ch2, wglr))
        gated2 = gates[:, :2 * H] * ch2
        cell = jnp.tanh(a[:, 5 * H:] + _dot(gated2, wclr))
        z2 = gates[:, 2 * H:4 * H] * ch2
        hidden = z2[:, :H] + z2[:, H:] + gates[:, 4 * H:] * cell
        if level > 0:
            out_ref[0, start - 1:start - 1 + n, :] = hidden
        else:
            out_ref[0, N - 1:N, :] = hidden
            outt_ref[0, :, :] = hidden


def kernel(inputs, W_gih, b_gih, W_glhh, W_grhh, W_cih, b_cih, W_clhh, W_crhh):
    wxc = jnp.concatenate([W_gih, W_cih], axis=0)
    bxc = jnp.concatenate([b_gih, b_cih]).reshape(1, 6 * H)
    wglr = jnp.concatenate([W_glhh, W_grhh], axis=1)
    wclr = jnp.concatenate([W_clhh, W_crhh], axis=1)

    outputs, output_t = pl.pallas_call(
        _tree_gru_kernel,
        grid=(B,),
        in_specs=[
            pl.BlockSpec(memory_space=pl.ANY),
            pl.BlockSpec((6 * H, IN_DIM), lambda b: (0, 0)),
            pl.BlockSpec((1, 6 * H), lambda b: (0, 0)),
            pl.BlockSpec((5 * H, 2 * H), lambda b: (0, 0)),
            pl.BlockSpec((H, 2 * H), lambda b: (0, 0)),
        ],
        out_specs=[
            pl.BlockSpec((1, N, H), lambda b: (b, 0, 0)),
            pl.BlockSpec((1, 1, H), lambda b: (b, 0, 0)),
        ],
        out_shape=[
            jax.ShapeDtypeStruct((B, N, H), jnp.float32),
            jax.ShapeDtypeStruct((B, 1, H), jnp.float32),
        ],
        scratch_shapes=[
            pltpu.VMEM((2, N, IN_DIM), jnp.float32),
            pltpu.SemaphoreType.DMA((2,)),
        ],
    )(inputs, wxc, bxc, wglr, wclr)
    return outputs, output_t.reshape(B, H)


# speedup vs baseline: 8.9143x; 1.0085x over previous
import jax
import jax.numpy as jnp
from jax.experimental import pallas as pl
from jax.experimental.pallas import tpu as pltpu

D = 12
N = 2 ** D - 1
B = 32
IN_DIM = 128
H = 128


def _sigmoid(v):
    return 0.5 * jnp.tanh(0.5 * v) + 0.5


def _dot(a, b):
    return jax.lax.dot_general(
        a, b, dimension_numbers=(((1,), (1,)), ((), ())),
        preferred_element_type=jnp.float32)


def _tree_gru_kernel(x_hbm, wxc_ref, bxc_ref, wglr_ref, wclr_ref,
                     out_ref, outt_ref, xbuf, sems):
    b = pl.program_id(0)

    def x_copy(slot, bidx):
        return pltpu.make_async_copy(
            x_hbm.at[:, bidx, :], xbuf.at[slot], sems.at[slot])

    @pl.when(b == 0)
    def _():
        x_copy(0, 0).start()

    @pl.when(b + 1 < B)
    def _():
        x_copy((b + 1) % 2, b + 1).start()

    slot = b % 2
    x_copy(slot, b).wait()
    x_ref = xbuf.at[slot]

    wxc = wxc_ref[...]
    bxc = bxc_ref[...]
    wglr = wglr_ref[...]
    wclr = wclr_ref[...]

    n = 2 ** (D - 1)
    start = n - 1
    x = x_ref[start:start + n, :]
    a = _dot(x, wxc[4 * H:, :]) + bxc[:, 4 * H:]
    hidden = _sigmoid(a[:, :H]) * jnp.tanh(a[:, H:])
    out_ref[0, start - 1:start - 1 + n, :] = hidden

    for level in range(D - 2, -1, -1):
        n = 2 ** level
        start = n - 1
        x = x_ref[start:start + n, :]
        ch = out_ref[0, 2 * start:2 * start + 2 * n, :]
        ch2 = ch.reshape(n, 2 * H)
        a = _dot(x, wxc) + bxc
        gates = _sigmoid(a[:, :5 * H] + _dot(ch2, wglr))
        gated2 = gates[:, :2 * H] * ch2
        cell = jnp.tanh(a[:, 5 * H:] + _dot(gated2, wclr))
        z2 = gates[:, 2 * H:4 * H] * ch2
        hidden = z2[:, :H] + z2[:, H:] + gates[:, 4 * H:] * cell
        if level > 0:
            out_ref[0, start - 1:start - 1 + n, :] = hidden
        else:
            out_ref[0, N - 1:N, :] = hidden
            outt_ref[0, :, :] = hidden


def kernel(inputs, W_gih, b_gih, W_glhh, W_grhh, W_cih, b_cih, W_clhh, W_crhh):
    wxc = jnp.concatenate([W_gih, W_cih], axis=0)
    bxc = jnp.concatenate([b_gih, b_cih]).reshape(1, 6 * H)
    wglr = jnp.concatenate([W_glhh, W_grhh], axis=1)
    wclr = jnp.concatenate([W_clhh, W_crhh], axis=1)

    outputs, output_t = pl.pallas_call(
        _tree_gru_kernel,
        grid=(B,),
        in_specs=[
            pl.BlockSpec(memory_space=pl.ANY),
            pl.BlockSpec((6 * H, IN_DIM), lambda b: (0, 0)),
            pl.BlockSpec((1, 6 * H), lambda b: (0, 0)),
            pl.BlockSpec((5 * H, 2 * H), lambda b: (0, 0)),
            pl.BlockSpec((H, 2 * H), lambda b: (0, 0)),
        ],
        out_specs=[
            pl.BlockSpec((1, N, H), lambda b: (b, 0, 0)),
            pl.BlockSpec((1, 1, H), lambda b: (b, 0, 0)),
        ],
        out_shape=[
            jax.ShapeDtypeStruct((B, N, H), jnp.float32),
            jax.ShapeDtypeStruct((B, 1, H), jnp.float32),
        ],
        scratch_shapes=[
            pltpu.VMEM((2, N, IN_DIM), jnp.float32),
            pltpu.SemaphoreType.DMA((2,)),
        ],
    )(inputs, wxc, bxc, wglr, wclr)
    return outputs, output_t.reshape(B, H)
